# baseline (speedup 1.0000x reference)
import jax
import jax.numpy as jnp
from jax.experimental import pallas as pl
from jax.experimental.pallas import tpu as pltpu

_B, _S, _V = 1024, 50, 1000
_BB = 8


def _tc_body(reward_ref, baseline_ref, lp_ref, seq_ref, out_ref, acc_ref):
    i = pl.program_id(0)

    @pl.when(i == 0)
    def _init():
        acc_ref[0] = 0.0
        acc_ref[1] = 0.0

    seq = seq_ref[...]
    adv = reward_ref[...] - baseline_ref[...]
    w = jnp.where(seq > 0, adv, 0.0)
    iota_v = jax.lax.broadcasted_iota(jnp.int32, (_BB, _V), 1)
    total = jnp.float32(0.0)
    for s in range(_S):
        lp2 = lp_ref[:, s, :]
        eq = iota_v == seq[:, s][:, None]
        total += jnp.sum(jnp.where(eq, lp2, 0.0) * w[:, s][:, None])
    acc_ref[0] += total
    acc_ref[1] += jnp.sum((seq > 0).astype(jnp.float32))

    @pl.when(i == pl.num_programs(0) - 1)
    def _fin():
        loss_sum = -acc_ref[0]
        cnt = acc_ref[1]
        out_ref[0, 0] = jnp.where(cnt > 0, loss_sum / cnt, loss_sum)


def kernel(reward, baseline, log_probs, seq):
    grid = (_B // _BB,)
    out = pl.pallas_call(
        _tc_body,
        grid=grid,
        in_specs=[
            pl.BlockSpec((_BB, 1), lambda i: (i, 0)),
            pl.BlockSpec((_BB, 1), lambda i: (i, 0)),
            pl.BlockSpec((_BB, _S, _V), lambda i: (i, 0, 0)),
            pl.BlockSpec((_BB, _S), lambda i: (i, 0)),
        ],
        out_specs=pl.BlockSpec(memory_space=pltpu.SMEM),
        out_shape=jax.ShapeDtypeStruct((1, 1), jnp.float32),
        scratch_shapes=[pltpu.SMEM((2,), jnp.float32)],
        compiler_params=pltpu.CompilerParams(
            dimension_semantics=("arbitrary",),
        ),
    )(reward, baseline, log_probs, seq)
    return out[0, 0]


# speedup vs baseline: 1.4828x; 1.4828x over previous
import jax
import jax.numpy as jnp
from jax.experimental import pallas as pl
from jax.experimental.pallas import tpu as pltpu

_B, _S, _V = 1024, 50, 1000
_BB = 8


def _tc_body(reward_ref, baseline_ref, lp_ref, seq_ref, out_ref,
             grand_ref, cnt_ref):
    i = pl.program_id(0)

    @pl.when(i == 0)
    def _init():
        grand_ref[...] = jnp.zeros_like(grand_ref)
        cnt_ref[...] = jnp.zeros_like(cnt_ref)

    seq = seq_ref[...]
    adv = reward_ref[...] - baseline_ref[...]
    pos = seq > 0
    w = jnp.where(pos, adv, 0.0)
    iota_v = jax.lax.broadcasted_iota(jnp.int32, (_BB, _V), 1)
    for s in range(_S):
        lp2 = lp_ref[:, s, :]
        eq = iota_v == seq[:, s][:, None]
        grand_ref[...] += jnp.where(eq, lp2, 0.0) * w[:, s][:, None]
    cnt_ref[...] += pos.astype(jnp.float32)

    @pl.when(i == pl.num_programs(0) - 1)
    def _fin():
        loss_sum = -jnp.sum(grand_ref[...])
        cnt = jnp.sum(cnt_ref[...])
        out_ref[0, 0] = jnp.where(cnt > 0, loss_sum / cnt, loss_sum)


def kernel(reward, baseline, log_probs, seq):
    grid = (_B // _BB,)
    out = pl.pallas_call(
        _tc_body,
        grid=grid,
        in_specs=[
            pl.BlockSpec((_BB, 1), lambda i: (i, 0)),
            pl.BlockSpec((_BB, 1), lambda i: (i, 0)),
            pl.BlockSpec((_BB, _S, _V), lambda i: (i, 0, 0)),
            pl.BlockSpec((_BB, _S), lambda i: (i, 0)),
        ],
        out_specs=pl.BlockSpec(memory_space=pltpu.SMEM),
        out_shape=jax.ShapeDtypeStruct((1, 1), jnp.float32),
        scratch_shapes=[
            pltpu.VMEM((_BB, _V), jnp.float32),
            pltpu.VMEM((_BB, _S), jnp.float32),
        ],
        compiler_params=pltpu.CompilerParams(
            dimension_semantics=("arbitrary",),
        ),
    )(reward, baseline, log_probs, seq)
    return out[0, 0]


# speedup vs baseline: 1.7278x; 1.1652x over previous
import jax
import jax.numpy as jnp
from jax.experimental import pallas as pl
from jax.experimental.pallas import tpu as pltpu

_B, _S, _V = 1024, 50, 1000
_BB = 32


def _tc_body(reward_ref, baseline_ref, lp_ref, seq_ref, out_ref,
             grand_ref, cnt_ref):
    i = pl.program_id(0)

    @pl.when(i == 0)
    def _init():
        grand_ref[...] = jnp.zeros_like(grand_ref)
        cnt_ref[...] = jnp.zeros_like(cnt_ref)

    seq = seq_ref[...]
    adv = reward_ref[...] - baseline_ref[...]
    pos = seq > 0
    w = jnp.where(pos, adv, 0.0)
    iota_v = jax.lax.broadcasted_iota(jnp.int32, (_BB, _V), 1)
    for s in range(_S):
        lp2 = lp_ref[:, s, :]
        eq = iota_v == seq[:, s][:, None]
        grand_ref[...] += jnp.where(eq, lp2, 0.0) * w[:, s][:, None]
    cnt_ref[...] += pos.astype(jnp.float32)

    @pl.when(i == pl.num_programs(0) - 1)
    def _fin():
        loss_sum = -jnp.sum(grand_ref[...])
        cnt = jnp.sum(cnt_ref[...])
        out_ref[0, 0] = jnp.where(cnt > 0, loss_sum / cnt, loss_sum)


def kernel(reward, baseline, log_probs, seq):
    grid = (_B // _BB,)
    out = pl.pallas_call(
        _tc_body,
        grid=grid,
        in_specs=[
            pl.BlockSpec((_BB, 1), lambda i: (i, 0)),
            pl.BlockSpec((_BB, 1), lambda i: (i, 0)),
            pl.BlockSpec((_BB, _S, _V), lambda i: (i, 0, 0)),
            pl.BlockSpec((_BB, _S), lambda i: (i, 0)),
        ],
        out_specs=pl.BlockSpec(memory_space=pltpu.SMEM),
        out_shape=jax.ShapeDtypeStruct((1, 1), jnp.float32),
        scratch_shapes=[
            pltpu.VMEM((_BB, _V), jnp.float32),
            pltpu.VMEM((_BB, _S), jnp.float32),
        ],
        compiler_params=pltpu.CompilerParams(
            dimension_semantics=("arbitrary",),
        ),
    )(reward, baseline, log_probs, seq)
    return out[0, 0]
